# baseline (speedup 1.0000x reference)
import functools

import jax
import jax.numpy as jnp
from jax import lax
from jax.experimental import pallas as pl
from jax.experimental.pallas import tpu as pltpu, tpu_sc as plsc

_NC = 2
_NS = 16
_HALF = 128
_K = 64
_NB = 3
_NI = 6


@functools.lru_cache(maxsize=None)
def _make_sc_agg(n, ep, with_counts):
    iters = ep // _K
    rps = n // _NS
    zr = 32 if rps % 32 == 0 else rps
    zcopies = rps // zr

    mesh = plsc.VectorSubcoreMesh(core_axis_name="c", subcore_axis_name="s")

    out_type = [
        jax.ShapeDtypeStruct((n, _HALF), jnp.float32),
        jax.ShapeDtypeStruct((n, _HALF), jnp.float32),
    ]
    scratch = [
        [pltpu.VMEM((2, _K), jnp.int32) for _ in range(_NI)],
        [pltpu.SemaphoreType.DMA for _ in range(_NI)],
        [pltpu.VMEM((_K, _HALF), jnp.float32) for _ in range(_NB)],
        [pltpu.SemaphoreType.DMA for _ in range(_NB)],
        [pltpu.SemaphoreType.DMA for _ in range(_NB)],
        pltpu.VMEM((zr, _HALF), jnp.float32),
        pltpu.SemaphoreType.DMA,
        pltpu.VMEM_SHARED((n, _HALF), jnp.float32),
    ]
    if with_counts:
        out_type += [
            jax.ShapeDtypeStruct((_NS, n), jnp.float32),
            jax.ShapeDtypeStruct((n,), jnp.float32),
        ]
        scratch += [
            pltpu.VMEM((n,), jnp.float32),
            pltpu.VMEM((rps,), jnp.float32),
            pltpu.VMEM((rps,), jnp.float32),
        ]

    def body(xlo, xhi, sd, *rest):
        if with_counts:
            (agg_lo, agg_hi, parts, cnt_out, slots, isem, rows, gsem, ssem,
             zb, zsem, acc, hist, rbuf, cbuf) = rest
        else:
            (agg_lo, agg_hi, slots, isem, rows, gsem, ssem,
             zb, zsem, acc) = rest
        c = lax.axis_index("c")
        s = lax.axis_index("s")
        r0 = s * rps

        for j in range(_NI):
            pltpu.async_copy(sd.at[s, j], slots[j], isem[j])

        @pl.loop(0, zr)
        def _(i):
            for j in range(_HALF // 16):
                zb[i, pl.ds(j * 16, 16)] = jnp.zeros((16,), jnp.float32)

        for t in range(zcopies):
            pltpu.async_copy(zb, acc.at[pl.ds(r0 + t * zr, zr)], zsem)
        for t in range(zcopies):
            pltpu.make_async_copy(zb, acc.at[pl.ds(r0 + t * zr, zr)],
                                  zsem).wait()

        if with_counts:
            @pl.when(c == 0)
            def _():
                @pl.loop(0, n // 16)
                def _(i):
                    hist[pl.ds(i * 16, 16)] = jnp.zeros((16,), jnp.float32)

        plsc.subcore_barrier()

        ones16 = jnp.ones((16,), jnp.float32)

        def run_edges(xh, with_hist):
            for i in range(2):
                pltpu.make_async_copy(sd.at[s, i], slots[i], isem[i]).wait()
                pltpu.async_copy(xh.at[slots[i].at[0]], rows[i], gsem[i])

            @pl.loop(0, iters, step=_NI)
            def _(i0):
                for u in range(_NI):
                    i = i0 + u
                    b = u % _NB
                    j = u
                    j2 = (u + 2) % _NI
                    b2 = (u + 2) % _NB
                    jr = (u + _NI - 1) % _NI
                    pltpu.make_async_copy(xh.at[slots[j].at[0]], rows[b],
                                          gsem[b]).wait()
                    pltpu.async_copy(rows[b], acc.at[slots[j].at[1]],
                                     ssem[b], add=True)
                    if with_hist:
                        for t in range(_K // 16):
                            idx16 = slots[j][1, pl.ds(t * 16, 16)]
                            plsc.addupdate_scatter(hist, [idx16], ones16)

                    @pl.when(i + 2 < iters)
                    def _():
                        @pl.when(i >= 1)
                        def _():
                            pltpu.make_async_copy(
                                rows[b2], acc.at[slots[jr].at[1]],
                                ssem[b2]).wait()
                        pltpu.make_async_copy(sd.at[s, i + 2], slots[j2],
                                              isem[j2]).wait()
                        pltpu.async_copy(xh.at[slots[j2].at[0]], rows[b2],
                                         gsem[b2])

                    @pl.when((i >= 1) & (i + 5 < iters))
                    def _():
                        pltpu.async_copy(sd.at[s, i + 5], slots[jr], isem[jr])

            for q in range(iters - _NB, iters):
                pltpu.make_async_copy(rows[q % _NB],
                                      acc.at[slots[q % _NI].at[1]],
                                      ssem[q % _NB]).wait()

        @pl.when(c == 0)
        def _():
            run_edges(xlo, with_counts)

        @pl.when(c == 1)
        def _():
            run_edges(xhi, False)

        if with_counts:
            @pl.when(c == 0)
            def _():
                pltpu.sync_copy(hist, parts.at[s])

        plsc.subcore_barrier()

        @pl.when(c == 0)
        def _():
            pltpu.sync_copy(acc.at[pl.ds(r0, rps)], agg_lo.at[pl.ds(r0, rps)])

        @pl.when(c == 1)
        def _():
            pltpu.sync_copy(acc.at[pl.ds(r0, rps)], agg_hi.at[pl.ds(r0, rps)])

        if with_counts:
            @pl.when(c == 0)
            def _():
                pltpu.sync_copy(parts.at[0, pl.ds(r0, rps)], cbuf)
                for r in range(1, _NS):
                    pltpu.sync_copy(parts.at[r, pl.ds(r0, rps)], rbuf)

                    @pl.loop(0, rps // 16)
                    def _(t):
                        sl = pl.ds(t * 16, 16)
                        cbuf[sl] = cbuf[sl] + rbuf[sl]

                pltpu.sync_copy(cbuf, cnt_out.at[pl.ds(r0, rps)])

    return pl.kernel(body, out_type=out_type, mesh=mesh,
                     scratch_types=scratch,
                     compiler_params=pltpu.CompilerParams(
                         needs_layout_passes=False),
                     name="sc_seg_mean" + ("_cnt" if with_counts else ""))


def _tc_layer(agg_lo_ref, agg_hi_ref, cnt_ref, x_lo_ref, x_hi_ref,
              a_ref, b_ref, bias_ref, *out_refs, relu):
    inv = 1.0 / jnp.maximum(cnt_ref[...], 1.0)
    m_lo = agg_lo_ref[...] * inv
    m_hi = agg_hi_ref[...] * inv
    acc = jnp.dot(m_lo, a_ref[0:_HALF, :], preferred_element_type=jnp.float32)
    acc = acc + jnp.dot(m_hi, a_ref[_HALF:, :],
                        preferred_element_type=jnp.float32)
    acc = acc + jnp.dot(x_lo_ref[...], b_ref[0:_HALF, :],
                        preferred_element_type=jnp.float32)
    acc = acc + jnp.dot(x_hi_ref[...], b_ref[_HALF:, :],
                        preferred_element_type=jnp.float32)
    acc = acc + bias_ref[...]
    if relu:
        h = jnp.maximum(acc, 0.0)
        out_refs[0][...] = h[:, :_HALF]
        out_refs[1][...] = h[:, _HALF:]
    else:
        out_refs[0][...] = acc


@functools.lru_cache(maxsize=None)
def _make_tc_layer(n, d, relu):
    br = 1000 if n % 1000 == 0 else n
    grid = (n // br,)
    row_spec = lambda w: pl.BlockSpec((br, w), lambda i: (i, 0))
    full_spec = lambda r, w: pl.BlockSpec((r, w), lambda i: (0, 0))
    in_specs = [
        row_spec(_HALF), row_spec(_HALF), row_spec(1),
        row_spec(_HALF), row_spec(_HALF),
        full_spec(d, d), full_spec(d, d), full_spec(1, d),
    ]
    if relu:
        out_shape = [jax.ShapeDtypeStruct((n, _HALF), jnp.float32)] * 2
        out_specs = [row_spec(_HALF), row_spec(_HALF)]
    else:
        out_shape = jax.ShapeDtypeStruct((n, d), jnp.float32)
        out_specs = row_spec(d)
    return pl.pallas_call(
        functools.partial(_tc_layer, relu=relu),
        grid=grid, in_specs=in_specs, out_specs=out_specs,
        out_shape=out_shape,
    )


def kernel(x, edge_index, W1l, b1l, W1r, W2l, b2l, W2r):
    n, d = x.shape
    e = edge_index.shape[1]
    src = edge_index[0]
    dst = edge_index[1]
    xlo = x[:, :_HALF]
    xhi = x[:, _HALF:]

    n_pad = -(-n // 640) * 640
    step = _NS * _K * _NI
    e_pad = -(-e // step) * step
    ep = e_pad // _NS
    if e_pad != e:
        pad = e_pad - e
        src = jnp.concatenate([src, jnp.zeros((pad,), jnp.int32)])
        dst = jnp.concatenate([dst, jnp.full((pad,), n_pad - 1, jnp.int32)])
    sd = jnp.stack([src.reshape(_NS, ep // _K, _K),
                    dst.reshape(_NS, ep // _K, _K)], axis=2)

    sc1 = _make_sc_agg(n_pad, ep, True)
    sc2 = _make_sc_agg(n_pad, ep, False)
    tc1 = _make_tc_layer(n, d, True)
    tc2 = _make_tc_layer(n, d, False)

    agg1lo, agg1hi, _parts, cnt = sc1(xlo, xhi, sd)
    cnt_col = cnt.reshape(n_pad, 1)
    hlo, hhi = tc1(agg1lo, agg1hi, cnt_col, xlo, xhi,
                   W1l.T, W1r.T, b1l.reshape(1, d))
    agg2lo, agg2hi = sc2(hlo, hhi, sd)
    out = tc2(agg2lo, agg2hi, cnt_col, hlo, hhi,
              W2l.T, W2r.T, b2l.reshape(1, d))
    return out


# speedup vs baseline: 1.3902x; 1.3902x over previous
import functools

import jax
import jax.numpy as jnp
from jax import lax
from jax.experimental import pallas as pl
from jax.experimental.pallas import tpu as pltpu, tpu_sc as plsc

_NC = 2
_NS = 16
_HALF = 128
_K = 128
_NB = 2
_NI = 4


@functools.lru_cache(maxsize=None)
def _make_sc_agg(n, ep):
    iters = ep // _K
    rps = n // _NS
    zr = 32 if rps % 32 == 0 else rps
    zcopies = rps // zr

    mesh = plsc.VectorSubcoreMesh(core_axis_name="c", subcore_axis_name="s")

    out_type = [
        jax.ShapeDtypeStruct((n, _HALF), jnp.float32),
        jax.ShapeDtypeStruct((n, _HALF), jnp.float32),
    ]
    scratch = [
        [pltpu.VMEM((2, _K), jnp.int32) for _ in range(_NI)],
        [pltpu.SemaphoreType.DMA for _ in range(_NI)],
        [pltpu.VMEM((_K, _HALF), jnp.float32) for _ in range(_NB)],
        [pltpu.SemaphoreType.DMA for _ in range(_NB)],
        pltpu.VMEM((zr, _HALF), jnp.float32),
        pltpu.SemaphoreType.DMA,
        pltpu.VMEM_SHARED((n, _HALF), jnp.float32),
    ]

    def body(xlo, xhi, sd, agg_lo, agg_hi, slots, isem, rows, gsem,
             zb, zsem, acc):
        c = lax.axis_index("c")
        s = lax.axis_index("s")
        r0 = s * rps

        for j in range(_NI):
            pltpu.async_copy(sd.at[s, j], slots[j], isem[j])

        @pl.loop(0, zr)
        def _(i):
            for j in range(_HALF // 16):
                zb[i, pl.ds(j * 16, 16)] = jnp.zeros((16,), jnp.float32)

        for t in range(zcopies):
            pltpu.async_copy(zb, acc.at[pl.ds(r0 + t * zr, zr)], zsem)
        for t in range(zcopies):
            pltpu.make_async_copy(zb, acc.at[pl.ds(r0 + t * zr, zr)],
                                  zsem).wait()

        plsc.subcore_barrier()

        def run_edges(xh):
            for i in range(_NB):
                pltpu.make_async_copy(sd.at[s, i], slots[i], isem[i]).wait()
                pltpu.async_copy(xh.at[slots[i].at[0]], rows[i], gsem[i])

            @pl.loop(0, iters, step=_NI)
            def _(i0):
                for u in range(_NI):
                    i = i0 + u
                    b = u % _NB
                    j = u
                    pltpu.make_async_copy(xh.at[slots[j].at[0]], rows[b],
                                          gsem[b]).wait()
                    pltpu.sync_copy(rows[b], acc.at[slots[j].at[1]], add=True)

                    @pl.when(i + _NI < iters)
                    def _():
                        pltpu.async_copy(sd.at[s, i + _NI], slots[j], isem[j])

                    @pl.when(i + _NB < iters)
                    def _():
                        j2 = (u + _NB) % _NI
                        pltpu.make_async_copy(sd.at[s, i + _NB], slots[j2],
                                              isem[j2]).wait()
                        pltpu.async_copy(xh.at[slots[j2].at[0]], rows[b],
                                         gsem[b])

        @pl.when(c == 0)
        def _():
            run_edges(xlo)

        @pl.when(c == 1)
        def _():
            run_edges(xhi)

        plsc.subcore_barrier()

        @pl.when(c == 0)
        def _():
            pltpu.sync_copy(acc.at[pl.ds(r0, rps)], agg_lo.at[pl.ds(r0, rps)])

        @pl.when(c == 1)
        def _():
            pltpu.sync_copy(acc.at[pl.ds(r0, rps)], agg_hi.at[pl.ds(r0, rps)])

    return pl.kernel(body, out_type=out_type, mesh=mesh,
                     scratch_types=scratch,
                     compiler_params=pltpu.CompilerParams(
                         needs_layout_passes=False),
                     name="sc_seg_sum")


@functools.lru_cache(maxsize=None)
def _make_sc_cnt(n, ep):
    rps = n // _NS

    mesh = plsc.VectorSubcoreMesh(core_axis_name="c", subcore_axis_name="s")

    scratch = [
        pltpu.VMEM((ep,), jnp.int32),
        pltpu.VMEM((n,), jnp.float32),
        pltpu.VMEM((rps,), jnp.float32),
        pltpu.VMEM((rps,), jnp.float32),
        pltpu.VMEM_SHARED((_NS, n), jnp.float32),
    ]

    def body(dstf, cnt_out, dstv, hist, rbuf, cbuf, shared):
        c = lax.axis_index("c")
        s = lax.axis_index("s")
        r0 = s * rps
        ones16 = jnp.ones((16,), jnp.float32)

        @pl.when(c == 0)
        def _():
            pltpu.sync_copy(dstf.at[pl.ds(s * ep, ep)], dstv)

            @pl.loop(0, n // 16)
            def _(i):
                hist[pl.ds(i * 16, 16)] = jnp.zeros((16,), jnp.float32)

            @pl.loop(0, ep // 16)
            def _(j):
                idx16 = dstv[pl.ds(j * 16, 16)]
                plsc.addupdate_scatter(hist, [idx16], ones16)

            pltpu.sync_copy(hist, shared.at[s])
            plsc.subcore_barrier()

            pltpu.sync_copy(shared.at[0, pl.ds(r0, rps)], cbuf)
            for r in range(1, _NS):
                pltpu.sync_copy(shared.at[r, pl.ds(r0, rps)], rbuf)

                @pl.loop(0, rps // 16)
                def _(t):
                    sl = pl.ds(t * 16, 16)
                    cbuf[sl] = cbuf[sl] + rbuf[sl]

            pltpu.sync_copy(cbuf, cnt_out.at[pl.ds(r0, rps)])

    return pl.kernel(body, out_type=jax.ShapeDtypeStruct((n,), jnp.float32),
                     mesh=mesh, scratch_types=scratch,
                     compiler_params=pltpu.CompilerParams(
                         needs_layout_passes=False),
                     name="sc_degree")


def _tc_layer(agg_lo_ref, agg_hi_ref, cnt_ref, x_lo_ref, x_hi_ref,
              a_ref, b_ref, bias_ref, *out_refs, relu):
    inv = 1.0 / jnp.maximum(cnt_ref[...], 1.0)
    m_lo = agg_lo_ref[...] * inv
    m_hi = agg_hi_ref[...] * inv
    acc = jnp.dot(m_lo, a_ref[0:_HALF, :], preferred_element_type=jnp.float32)
    acc = acc + jnp.dot(m_hi, a_ref[_HALF:, :],
                        preferred_element_type=jnp.float32)
    acc = acc + jnp.dot(x_lo_ref[...], b_ref[0:_HALF, :],
                        preferred_element_type=jnp.float32)
    acc = acc + jnp.dot(x_hi_ref[...], b_ref[_HALF:, :],
                        preferred_element_type=jnp.float32)
    acc = acc + bias_ref[...]
    if relu:
        h = jnp.maximum(acc, 0.0)
        out_refs[0][...] = h[:, :_HALF]
        out_refs[1][...] = h[:, _HALF:]
    else:
        out_refs[0][...] = acc


@functools.lru_cache(maxsize=None)
def _make_tc_layer(n, d, relu):
    br = 1000 if n % 1000 == 0 else n
    grid = (n // br,)
    row_spec = lambda w: pl.BlockSpec((br, w), lambda i: (i, 0))
    full_spec = lambda r, w: pl.BlockSpec((r, w), lambda i: (0, 0))
    in_specs = [
        row_spec(_HALF), row_spec(_HALF), row_spec(1),
        row_spec(_HALF), row_spec(_HALF),
        full_spec(d, d), full_spec(d, d), full_spec(1, d),
    ]
    if relu:
        out_shape = [jax.ShapeDtypeStruct((n, _HALF), jnp.float32)] * 2
        out_specs = [row_spec(_HALF), row_spec(_HALF)]
    else:
        out_shape = jax.ShapeDtypeStruct((n, d), jnp.float32)
        out_specs = row_spec(d)
    return pl.pallas_call(
        functools.partial(_tc_layer, relu=relu),
        grid=grid, in_specs=in_specs, out_specs=out_specs,
        out_shape=out_shape,
    )


def kernel(x, edge_index, W1l, b1l, W1r, W2l, b2l, W2r):
    n, d = x.shape
    e = edge_index.shape[1]
    src = edge_index[0]
    dst = edge_index[1]
    xlo = x[:, :_HALF]
    xhi = x[:, _HALF:]

    n_pad = -(-n // 640) * 640
    step = _NS * _K * _NI
    e_pad = -(-e // step) * step
    ep = e_pad // _NS
    if e_pad != e:
        pad = e_pad - e
        src = jnp.concatenate([src, jnp.zeros((pad,), jnp.int32)])
        dst = jnp.concatenate([dst, jnp.full((pad,), n_pad - 1, jnp.int32)])
    sd = jnp.stack([src.reshape(_NS, ep // _K, _K),
                    dst.reshape(_NS, ep // _K, _K)], axis=2)

    sc_agg = _make_sc_agg(n_pad, ep)
    sc_cnt = _make_sc_cnt(n_pad, ep)
    tc1 = _make_tc_layer(n, d, True)
    tc2 = _make_tc_layer(n, d, False)

    cnt = sc_cnt(dst)
    agg1lo, agg1hi = sc_agg(xlo, xhi, sd)
    cnt_col = cnt.reshape(n_pad, 1)
    hlo, hhi = tc1(agg1lo, agg1hi, cnt_col, xlo, xhi,
                   W1l.T, W1r.T, b1l.reshape(1, d))
    agg2lo, agg2hi = sc_agg(hlo, hhi, sd)
    out = tc2(agg2lo, agg2hi, cnt_col, hlo, hhi,
              W2l.T, W2r.T, b2l.reshape(1, d))
    return out


# speedup vs baseline: 2.1626x; 1.5556x over previous
import functools

import jax
import jax.numpy as jnp
from jax import lax
from jax.experimental import pallas as pl
from jax.experimental.pallas import tpu as pltpu, tpu_sc as plsc

_NC = 2
_NS = 16
_HALF = 128
_K = 128
_NB = 2
_NI = 4


@functools.lru_cache(maxsize=None)
def _make_sc_agg(n, ep, msg_bf16=True):
    iters = ep // _K
    rps = n // _NS
    zr = 32 if rps % 32 == 0 else rps
    zcopies = rps // zr
    dt = jnp.bfloat16 if msg_bf16 else jnp.float32
    lanes = 32 if msg_bf16 else 16

    mesh = plsc.VectorSubcoreMesh(core_axis_name="c", subcore_axis_name="s")

    out_type = [
        jax.ShapeDtypeStruct((n, _HALF), dt),
        jax.ShapeDtypeStruct((n, _HALF), dt),
    ]
    scratch = [
        [pltpu.VMEM((2, _K), jnp.int32) for _ in range(_NI)],
        [pltpu.SemaphoreType.DMA for _ in range(_NI)],
        [pltpu.VMEM((_K, _HALF), dt) for _ in range(_NB)],
        [pltpu.SemaphoreType.DMA for _ in range(_NB)],
        pltpu.VMEM((zr, _HALF), dt),
        pltpu.SemaphoreType.DMA,
        pltpu.VMEM_SHARED((n, _HALF), dt),
    ]

    def body(xlo, xhi, sd, agg_lo, agg_hi, slots, isem, rows, gsem,
             zb, zsem, acc):
        c = lax.axis_index("c")
        s = lax.axis_index("s")
        r0 = s * rps

        for j in range(_NI):
            pltpu.async_copy(sd.at[s, j], slots[j], isem[j])

        @pl.loop(0, zr)
        def _(i):
            for j in range(_HALF // lanes):
                zb[i, pl.ds(j * lanes, lanes)] = jnp.zeros((lanes,), dt)

        for t in range(zcopies):
            pltpu.async_copy(zb, acc.at[pl.ds(r0 + t * zr, zr)], zsem)
        for t in range(zcopies):
            pltpu.make_async_copy(zb, acc.at[pl.ds(r0 + t * zr, zr)],
                                  zsem).wait()

        plsc.subcore_barrier()

        def run_edges(xh):
            for i in range(_NB):
                pltpu.make_async_copy(sd.at[s, i], slots[i], isem[i]).wait()
                pltpu.async_copy(xh.at[slots[i].at[0]], rows[i], gsem[i])

            @pl.loop(0, iters, step=_NI)
            def _(i0):
                for u in range(_NI):
                    i = i0 + u
                    b = u % _NB
                    j = u
                    pltpu.make_async_copy(xh.at[slots[j].at[0]], rows[b],
                                          gsem[b]).wait()
                    pltpu.sync_copy(rows[b], acc.at[slots[j].at[1]], add=True)

                    @pl.when(i + _NI < iters)
                    def _():
                        pltpu.async_copy(sd.at[s, i + _NI], slots[j], isem[j])

                    @pl.when(i + _NB < iters)
                    def _():
                        j2 = (u + _NB) % _NI
                        pltpu.make_async_copy(sd.at[s, i + _NB], slots[j2],
                                              isem[j2]).wait()
                        pltpu.async_copy(xh.at[slots[j2].at[0]], rows[b],
                                         gsem[b])

        @pl.when(c == 0)
        def _():
            run_edges(xlo)

        @pl.when(c == 1)
        def _():
            run_edges(xhi)

        plsc.subcore_barrier()

        @pl.when(c == 0)
        def _():
            pltpu.sync_copy(acc.at[pl.ds(r0, rps)], agg_lo.at[pl.ds(r0, rps)])

        @pl.when(c == 1)
        def _():
            pltpu.sync_copy(acc.at[pl.ds(r0, rps)], agg_hi.at[pl.ds(r0, rps)])

    return pl.kernel(body, out_type=out_type, mesh=mesh,
                     scratch_types=scratch,
                     compiler_params=pltpu.CompilerParams(
                         needs_layout_passes=False,
                         use_tc_tiling_on_sc=False),
                     name="sc_seg_sum")


@functools.lru_cache(maxsize=None)
def _make_sc_cnt(n, ep):
    rps = n // _NS

    mesh = plsc.VectorSubcoreMesh(core_axis_name="c", subcore_axis_name="s")

    scratch = [
        pltpu.VMEM((ep,), jnp.int32),
        pltpu.VMEM((n,), jnp.float32),
        pltpu.VMEM((rps,), jnp.float32),
        pltpu.VMEM((rps,), jnp.float32),
        pltpu.VMEM_SHARED((_NS, n), jnp.float32),
    ]

    def body(dstf, cnt_out, dstv, hist, rbuf, cbuf, shared):
        c = lax.axis_index("c")
        s = lax.axis_index("s")
        r0 = s * rps
        ones16 = jnp.ones((16,), jnp.float32)

        @pl.when(c == 0)
        def _():
            pltpu.sync_copy(dstf.at[pl.ds(s * ep, ep)], dstv)

            @pl.loop(0, n // 16)
            def _(i):
                hist[pl.ds(i * 16, 16)] = jnp.zeros((16,), jnp.float32)

            @pl.loop(0, ep // 16)
            def _(j):
                idx16 = dstv[pl.ds(j * 16, 16)]
                plsc.addupdate_scatter(hist, [idx16], ones16)

            pltpu.sync_copy(hist, shared.at[s])
            plsc.subcore_barrier()

            pltpu.sync_copy(shared.at[0, pl.ds(r0, rps)], cbuf)
            for r in range(1, _NS):
                pltpu.sync_copy(shared.at[r, pl.ds(r0, rps)], rbuf)

                @pl.loop(0, rps // 16)
                def _(t):
                    sl = pl.ds(t * 16, 16)
                    cbuf[sl] = cbuf[sl] + rbuf[sl]

            pltpu.sync_copy(cbuf, cnt_out.at[pl.ds(r0, rps)])

    return pl.kernel(body, out_type=jax.ShapeDtypeStruct((n,), jnp.float32),
                     mesh=mesh, scratch_types=scratch,
                     compiler_params=pltpu.CompilerParams(
                         needs_layout_passes=False),
                     name="sc_degree")


def _tc_layer(agg_lo_ref, agg_hi_ref, cnt_ref, x_lo_ref, x_hi_ref,
              a_ref, b_ref, bias_ref, *out_refs, relu):
    inv = 1.0 / jnp.maximum(cnt_ref[...], 1.0)
    m_lo = agg_lo_ref[...].astype(jnp.float32) * inv
    m_hi = agg_hi_ref[...].astype(jnp.float32) * inv
    x_lo = x_lo_ref[...].astype(jnp.float32)
    x_hi = x_hi_ref[...].astype(jnp.float32)
    acc = jnp.dot(m_lo, a_ref[0:_HALF, :], preferred_element_type=jnp.float32)
    acc = acc + jnp.dot(m_hi, a_ref[_HALF:, :],
                        preferred_element_type=jnp.float32)
    acc = acc + jnp.dot(x_lo, b_ref[0:_HALF, :],
                        preferred_element_type=jnp.float32)
    acc = acc + jnp.dot(x_hi, b_ref[_HALF:, :],
                        preferred_element_type=jnp.float32)
    acc = acc + bias_ref[...]
    if relu:
        h = jnp.maximum(acc, 0.0)
        out_refs[0][...] = h[:, :_HALF].astype(jnp.bfloat16)
        out_refs[1][...] = h[:, _HALF:].astype(jnp.bfloat16)
    else:
        out_refs[0][...] = acc


@functools.lru_cache(maxsize=None)
def _make_tc_layer(n, d, relu):
    br = 1000 if n % 1000 == 0 else n
    grid = (n // br,)
    row_spec = lambda w: pl.BlockSpec((br, w), lambda i: (i, 0))
    full_spec = lambda r, w: pl.BlockSpec((r, w), lambda i: (0, 0))
    in_specs = [
        row_spec(_HALF), row_spec(_HALF), row_spec(1),
        row_spec(_HALF), row_spec(_HALF),
        full_spec(d, d), full_spec(d, d), full_spec(1, d),
    ]
    if relu:
        out_shape = [jax.ShapeDtypeStruct((n, _HALF), jnp.bfloat16)] * 2
        out_specs = [row_spec(_HALF), row_spec(_HALF)]
    else:
        out_shape = jax.ShapeDtypeStruct((n, d), jnp.float32)
        out_specs = row_spec(d)
    return pl.pallas_call(
        functools.partial(_tc_layer, relu=relu),
        grid=grid, in_specs=in_specs, out_specs=out_specs,
        out_shape=out_shape,
    )


def kernel(x, edge_index, W1l, b1l, W1r, W2l, b2l, W2r):
    n, d = x.shape
    e = edge_index.shape[1]
    src = edge_index[0]
    dst = edge_index[1]
    xlo = x[:, :_HALF]
    xhi = x[:, _HALF:]

    n_pad = -(-n // 640) * 640
    step = _NS * _K * _NI
    e_pad = -(-e // step) * step
    ep = e_pad // _NS
    if e_pad != e:
        pad = e_pad - e
        src = jnp.concatenate([src, jnp.zeros((pad,), jnp.int32)])
        dst = jnp.concatenate([dst, jnp.full((pad,), n_pad - 1, jnp.int32)])
    sd = jnp.stack([src.reshape(_NS, ep // _K, _K),
                    dst.reshape(_NS, ep // _K, _K)], axis=2)

    sc_agg = _make_sc_agg(n_pad, ep)
    sc_cnt = _make_sc_cnt(n_pad, ep)
    tc1 = _make_tc_layer(n, d, True)
    tc2 = _make_tc_layer(n, d, False)

    cnt = sc_cnt(dst)
    agg1lo, agg1hi = sc_agg(xlo.astype(jnp.bfloat16),
                            xhi.astype(jnp.bfloat16), sd)
    cnt_col = cnt.reshape(n_pad, 1)
    hlo, hhi = tc1(agg1lo, agg1hi, cnt_col, xlo, xhi,
                   W1l.T, W1r.T, b1l.reshape(1, d))
    agg2lo, agg2hi = sc_agg(hlo, hhi, sd)
    out = tc2(agg2lo, agg2hi, cnt_col, hlo, hhi,
              W2l.T, W2r.T, b2l.reshape(1, d))
    return out
